# baseline (speedup 1.0000x reference)
import functools
import math

import jax
import jax.numpy as jnp
from jax import lax
from jax.experimental import pallas as pl
from jax.experimental.pallas import tpu as pltpu
from jax.experimental.pallas import tpu_sc as plsc

D_MODEL = 64
SCALE = math.sqrt(D_MODEL)

IDX_ROW = 128
STREAMS_PER_CHUNK = 4
CHUNK = IDX_ROW * STREAMS_PER_CHUNK


@functools.partial(jax.jit, static_argnums=(2, 3, 4))
def _sc_embed(x_flat3, table, nw, b_per_w, n_chunks):
    B = nw * b_per_w
    mesh = plsc.VectorSubcoreMesh(core_axis_name="c", subcore_axis_name="s")
    num_cores = 2

    @functools.partial(
        pl.kernel,
        out_type=jax.ShapeDtypeStruct((B, D_MODEL), jnp.float32),
        mesh=mesh,
        compiler_params=pltpu.CompilerParams(use_tc_tiling_on_sc=False),
        scratch_types=[
            pltpu.VMEM((b_per_w // IDX_ROW, IDX_ROW), jnp.int32),
            pltpu.VMEM((CHUNK, D_MODEL), jnp.float32),
            pltpu.SemaphoreType.DMA,
        ],
    )
    def body(x_hbm, tbl_hbm, out_hbm, idx_v, rows_v, gsem):
        wid = lax.axis_index("s") * num_cores + lax.axis_index("c")
        base = wid * b_per_w
        pltpu.sync_copy(x_hbm.at[wid], idx_v)

        def chunk_body(g, _):
            cps = []
            for k in range(STREAMS_PER_CHUNK):
                cps.append(
                    pltpu.async_copy(
                        tbl_hbm.at[idx_v.at[g * STREAMS_PER_CHUNK + k]],
                        rows_v.at[pl.ds(k * IDX_ROW, IDX_ROW)],
                        gsem,
                    )
                )
            for cp in cps:
                cp.wait()

            def scale_row(r, _):
                for c in range(D_MODEL // 16):
                    sl = pl.ds(c * 16, 16)
                    rows_v[r, sl] = rows_v[r, sl] * SCALE
                return _

            lax.fori_loop(0, CHUNK, scale_row, None, unroll=False)
            pltpu.sync_copy(rows_v, out_hbm.at[pl.ds(base + g * CHUNK, CHUNK)])
            return _

        lax.fori_loop(0, n_chunks, chunk_body, None, unroll=False)

    return body(x_flat3, table)


def kernel(x, table):
    B = x.shape[0] * x.shape[1]
    info = plsc.get_sparse_core_info()
    nw = info.num_cores * info.num_subcores
    b_per_w = B // nw
    n_chunks = b_per_w // CHUNK
    x3 = x.reshape(nw, b_per_w // IDX_ROW, IDX_ROW)
    out = _sc_embed(x3, table, nw, b_per_w, n_chunks)
    return out.reshape(x.shape[0], x.shape[1], D_MODEL)


# speedup vs baseline: 1.1169x; 1.1169x over previous
import functools
import math

import jax
import jax.numpy as jnp
from jax import lax
from jax.experimental import pallas as pl
from jax.experimental.pallas import tpu as pltpu
from jax.experimental.pallas import tpu_sc as plsc

D_MODEL = 64
SCALE = math.sqrt(D_MODEL)

IDX_ROW = 128
STREAMS_PER_CHUNK = 2
CHUNK = IDX_ROW * STREAMS_PER_CHUNK
ROWS_PER_ITER = 8


@functools.partial(jax.jit, static_argnums=(2, 3, 4))
def _sc_embed(x_flat3, table, nw, b_per_w, n_chunks):
    B = nw * b_per_w
    mesh = plsc.VectorSubcoreMesh(core_axis_name="c", subcore_axis_name="s")
    num_cores = 2

    @functools.partial(
        pl.kernel,
        out_type=jax.ShapeDtypeStruct((B, D_MODEL), jnp.float32),
        mesh=mesh,
        compiler_params=pltpu.CompilerParams(use_tc_tiling_on_sc=False),
        scratch_types=[
            pltpu.VMEM((b_per_w // IDX_ROW, IDX_ROW), jnp.int32),
            pltpu.VMEM((2, CHUNK, D_MODEL), jnp.float32),
            pltpu.VMEM((2, CHUNK, D_MODEL), jnp.float32),
            pltpu.SemaphoreType.DMA,
            pltpu.SemaphoreType.DMA,
        ],
    )
    def body(x_hbm, tbl_hbm, out_hbm, idx_v, grow_v, wrow_v, gsem, wsem):
        wid = lax.axis_index("s") * num_cores + lax.axis_index("c")
        base = wid * b_per_w
        pltpu.sync_copy(x_hbm.at[wid], idx_v)

        def gather_desc(g, slot):
            cps = []
            for k in range(STREAMS_PER_CHUNK):
                cps.append(
                    pltpu.make_async_copy(
                        tbl_hbm.at[idx_v.at[g * STREAMS_PER_CHUNK + k]],
                        grow_v.at[slot, pl.ds(k * IDX_ROW, IDX_ROW)],
                        gsem,
                    )
                )
            return cps

        def write_desc(g, slot):
            return pltpu.make_async_copy(
                wrow_v.at[slot],
                out_hbm.at[pl.ds(base + g * CHUNK, CHUNK)],
                wsem,
            )

        for b in range(2):
            for cp in gather_desc(b, b):
                cp.start()

        def step(g, slot):
            for cp in gather_desc(g, slot):
                cp.wait()
            pl.when(g >= 2)(lambda: write_desc(g - 2, slot).wait())

            def scale_iter(i, _):
                r0 = i * ROWS_PER_ITER
                for r in range(ROWS_PER_ITER):
                    for c in range(D_MODEL // 16):
                        sl = pl.ds(c * 16, 16)
                        wrow_v[slot, r0 + r, sl] = grow_v[slot, r0 + r, sl] * SCALE
                return _

            lax.fori_loop(0, CHUNK // ROWS_PER_ITER, scale_iter, None)
            write_desc(g, slot).start()

            def prefetch():
                for cp in gather_desc(g + 2, slot):
                    cp.start()

            pl.when(g + 2 < n_chunks)(prefetch)

        def pair(i, _):
            for b in range(2):
                step(2 * i + b, b)
            return _

        lax.fori_loop(0, n_chunks // 2, pair, None)
        write_desc(n_chunks - 2, 0).wait()
        write_desc(n_chunks - 1, 1).wait()

    return body(x_flat3, table)


def kernel(x, table):
    B = x.shape[0] * x.shape[1]
    info = plsc.get_sparse_core_info()
    nw = info.num_cores * info.num_subcores
    b_per_w = B // nw
    n_chunks = b_per_w // CHUNK
    x3 = x.reshape(nw, b_per_w // IDX_ROW, IDX_ROW)
    out = _sc_embed(x3, table, nw, b_per_w, n_chunks)
    return out.reshape(x.shape[0], x.shape[1], D_MODEL)
